# baseline (speedup 1.0000x reference)
import functools
import numpy as np
import jax
import jax.numpy as jnp
from jax import lax
from jax.experimental import pallas as pl
from jax.experimental.pallas import tpu as pltpu
from jax.experimental.pallas import tpu_sc as plsc

_N = 10000
_E = 320000
_F = 128
_H = 64
_L = 64
_ALPHA = 0.1
_THETA = 0.6

_NC = 2
_NS = 16
_NW = _NC * _NS
_EPW = _E // _NW
_CH = 200
_NCHUNK = _EPW // _CH
_NBUF = 2
_RPT = 624
_TAIL0 = _NS * _RPT
_TAILN = _N - _TAIL0

_mesh = plsc.VectorSubcoreMesh(core_axis_name="c", subcore_axis_name="s")


@functools.partial(
    pl.kernel,
    out_type=jax.ShapeDtypeStruct((_NC, _N, _H), jnp.float32),
    mesh=_mesh,
    scratch_types=[
        pltpu.VMEM((_NCHUNK, _CH), jnp.int32),
        pltpu.VMEM((_NCHUNK, _CH), jnp.int32),
        [pltpu.VMEM((_CH, _H), jnp.float32) for _ in range(_NBUF)],
        pltpu.VMEM_SHARED((_N, _H), jnp.float32),
        pltpu.VMEM_SHARED((_N, _H), jnp.float32),
        [pltpu.SemaphoreType.DMA for _ in range(_NBUF)],
        [pltpu.SemaphoreType.DMA for _ in range(_NBUF)],
    ],
    compiler_params=pltpu.CompilerParams(use_tc_tiling_on_sc=False),
)
def _segsum_sc(g_hbm, src_hbm, dst_hbm, zero_hbm, out_hbm,
               src_v, dst_v, rows, acc, gsp, gsem, ssem):
    cid = lax.axis_index("c")
    sid = lax.axis_index("s")
    wid = sid * _NC + cid
    r0 = sid * _RPT
    pltpu.sync_copy(zero_hbm.at[pl.ds(r0, _RPT)], acc.at[pl.ds(r0, _RPT)])
    pltpu.sync_copy(g_hbm.at[pl.ds(r0, _RPT)], gsp.at[pl.ds(r0, _RPT)])

    @pl.when(sid == _NS - 1)
    def _zero_tail():
        pltpu.sync_copy(zero_hbm.at[pl.ds(_TAIL0, _TAILN)],
                        acc.at[pl.ds(_TAIL0, _TAILN)])
        pltpu.sync_copy(g_hbm.at[pl.ds(_TAIL0, _TAILN)],
                        gsp.at[pl.ds(_TAIL0, _TAILN)])

    pltpu.sync_copy(src_hbm.at[wid], src_v)
    pltpu.sync_copy(dst_hbm.at[wid], dst_v)
    plsc.subcore_barrier()

    def start_gather(j, buf, sem):
        pltpu.async_copy(gsp.at[src_v.at[j]], buf, sem)

    def wait_gather(buf, sem):
        pltpu.make_async_copy(gsp.at[src_v.at[0]], buf, sem).wait()

    def start_scatter(j, buf, sem):
        pltpu.async_copy(buf, acc.at[dst_v.at[j]], sem, add=True)

    def wait_scatter(buf, sem):
        pltpu.make_async_copy(buf, acc.at[dst_v.at[0]], sem).wait()

    for b in range(_NBUF):
        start_gather(b, rows[b], gsem[b])

    def body(i, carry):
        j = i * _NBUF
        for b in range(_NBUF):
            wait_gather(rows[b], gsem[b])
            start_scatter(j + b, rows[b], ssem[b])
        for b in range(_NBUF):
            wait_scatter(rows[b], ssem[b])

            @pl.when(j + _NBUF + b < _NCHUNK)
            def _g(b=b, j=j):
                start_gather(j + _NBUF + b, rows[b], gsem[b])

        return carry

    lax.fori_loop(0, _NCHUNK // _NBUF, body, 0)
    plsc.subcore_barrier()
    pltpu.sync_copy(acc.at[pl.ds(r0, _RPT)], out_hbm.at[cid, pl.ds(r0, _RPT)])

    @pl.when(sid == _NS - 1)
    def _out_tail():
        pltpu.sync_copy(acc.at[pl.ds(_TAIL0, _TAILN)],
                        out_hbm.at[cid, pl.ds(_TAIL0, _TAILN)])


_RB = 2000
_GRID = _N // _RB


def _prologue_body(x_ref, w1_ref, b1_ref, p_ref,
                   h_ref, g_ref, dinv_ref):
    deg = p_ref[0, :, :1] + p_ref[1, :, :1] + 1.0
    dinv = lax.rsqrt(deg)
    h = jnp.dot(x_ref[...], w1_ref[...],
                preferred_element_type=jnp.float32,
                precision=lax.Precision.HIGHEST) + b1_ref[...]
    h = jnp.maximum(h, 0.0)
    h_ref[...] = h
    g_ref[...] = dinv * h
    dinv_ref[...] = dinv


_prologue = pl.pallas_call(
    _prologue_body,
    grid=(_GRID,),
    in_specs=[
        pl.BlockSpec((_RB, _F), lambda i: (i, 0)),
        pl.BlockSpec((_F, _H), lambda i: (0, 0)),
        pl.BlockSpec((1, _H), lambda i: (0, 0)),
        pl.BlockSpec((_NC, _RB, _H), lambda i: (0, i, 0)),
    ],
    out_specs=[
        pl.BlockSpec((_RB, _H), lambda i: (i, 0)),
        pl.BlockSpec((_RB, _H), lambda i: (i, 0)),
        pl.BlockSpec((_RB, 1), lambda i: (i, 0)),
    ],
    out_shape=[
        jax.ShapeDtypeStruct((_N, _H), jnp.float32),
        jax.ShapeDtypeStruct((_N, _H), jnp.float32),
        jax.ShapeDtypeStruct((_N, 1), jnp.float32),
    ],
)


def _support(dinv_ref, s_ref, g_ref, h0_ref):
    ax = dinv_ref[...] * (s_ref[0] + s_ref[1] + g_ref[...])
    return (1.0 - _ALPHA) * ax + _ALPHA * h0_ref[...]


def _layer_body(dinv_ref, s_ref, g_ref, h0_ref, m_ref, gout_ref):
    h = jnp.dot(_support(dinv_ref, s_ref, g_ref, h0_ref), m_ref[...],
                preferred_element_type=jnp.float32,
                precision=lax.Precision.HIGHEST)
    gout_ref[...] = dinv_ref[...] * jnp.maximum(h, 0.0)


_layer = pl.pallas_call(
    _layer_body,
    grid=(_GRID,),
    in_specs=[
        pl.BlockSpec((_RB, 1), lambda i: (i, 0)),
        pl.BlockSpec((_NC, _RB, _H), lambda i: (0, i, 0)),
        pl.BlockSpec((_RB, _H), lambda i: (i, 0)),
        pl.BlockSpec((_RB, _H), lambda i: (i, 0)),
        pl.BlockSpec((_H, _H), lambda i: (0, 0)),
    ],
    out_specs=pl.BlockSpec((_RB, _H), lambda i: (i, 0)),
    out_shape=jax.ShapeDtypeStruct((_N, _H), jnp.float32),
)


def _final_body(dinv_ref, s_ref, g_ref, h0_ref, m_ref, w2_ref, b2_ref,
                out_ref):
    h = jnp.dot(_support(dinv_ref, s_ref, g_ref, h0_ref), m_ref[...],
                preferred_element_type=jnp.float32,
                precision=lax.Precision.HIGHEST)
    h = jnp.maximum(h, 0.0)
    out_ref[...] = jnp.dot(h, w2_ref[...],
                           preferred_element_type=jnp.float32,
                           precision=lax.Precision.HIGHEST) + b2_ref[...]


_final = pl.pallas_call(
    _final_body,
    grid=(_GRID,),
    in_specs=[
        pl.BlockSpec((_RB, 1), lambda i: (i, 0)),
        pl.BlockSpec((_NC, _RB, _H), lambda i: (0, i, 0)),
        pl.BlockSpec((_RB, _H), lambda i: (i, 0)),
        pl.BlockSpec((_RB, _H), lambda i: (i, 0)),
        pl.BlockSpec((_H, _H), lambda i: (0, 0)),
        pl.BlockSpec((_H, _H), lambda i: (0, 0)),
        pl.BlockSpec((1, _H), lambda i: (0, 0)),
    ],
    out_specs=pl.BlockSpec((_RB, _H), lambda i: (i, 0)),
    out_shape=jax.ShapeDtypeStruct((_N, _H), jnp.float32),
)


def kernel(x, edges, W1, b1, conv_w, W2, b2):
    src2 = edges[0].reshape(_NW, _NCHUNK, _CH)
    dst2 = edges[1].reshape(_NW, _NCHUNK, _CH)
    zeros = jnp.zeros((_N, _H), jnp.float32)
    ones = jnp.ones((_N, _H), jnp.float32)

    beta = np.log(_THETA / np.arange(1, _L + 1) + 1.0).astype(np.float32)
    eye = jnp.eye(_H, dtype=jnp.float32)
    M = (1.0 - beta)[:, None, None] * eye + beta[:, None, None] * conv_w

    p = _segsum_sc(ones, src2, dst2, zeros)
    h0, g, dinv = _prologue(x, W1, b1.reshape(1, _H), p)
    for l in range(_L - 1):
        s = _segsum_sc(g, src2, dst2, zeros)
        g = _layer(dinv, s, g, h0, M[l])
    s = _segsum_sc(g, src2, dst2, zeros)
    return _final(dinv, s, g, h0, M[_L - 1], W2, b2.reshape(1, _H))


# speedup vs baseline: 1.0790x; 1.0790x over previous
import functools
import numpy as np
import jax
import jax.numpy as jnp
from jax import lax
from jax.experimental import pallas as pl
from jax.experimental.pallas import tpu as pltpu
from jax.experimental.pallas import tpu_sc as plsc

_N = 10000
_E = 320000
_F = 128
_H = 64
_L = 64
_ALPHA = 0.1
_THETA = 0.6

_NC = 2
_NS = 16
_NW = _NC * _NS
_EPW = _E // _NW
_CH = 500
_NCHUNK = _EPW // _CH
_NBUF = 2
_RPT = 624
_TAIL0 = _NS * _RPT
_TAILN = _N - _TAIL0

_mesh = plsc.VectorSubcoreMesh(core_axis_name="c", subcore_axis_name="s")


@functools.partial(
    pl.kernel,
    out_type=jax.ShapeDtypeStruct((_NC, _N, _H), jnp.float32),
    mesh=_mesh,
    scratch_types=[
        pltpu.VMEM((_NCHUNK, _CH), jnp.int32),
        pltpu.VMEM((_NCHUNK, _CH), jnp.int32),
        [pltpu.VMEM((_CH, _H), jnp.float32) for _ in range(_NBUF)],
        pltpu.VMEM_SHARED((_N, _H), jnp.float32),
        [pltpu.SemaphoreType.DMA for _ in range(_NBUF)],
        [pltpu.SemaphoreType.DMA for _ in range(_NBUF)],
    ],
    compiler_params=pltpu.CompilerParams(use_tc_tiling_on_sc=False),
)
def _segsum_sc(g_hbm, src_hbm, dst_hbm, zero_hbm, out_hbm,
               src_v, dst_v, rows, acc, gsem, ssem):
    cid = lax.axis_index("c")
    sid = lax.axis_index("s")
    wid = sid * _NC + cid
    r0 = sid * _RPT
    pltpu.sync_copy(zero_hbm.at[pl.ds(r0, _RPT)], acc.at[pl.ds(r0, _RPT)])

    @pl.when(sid == _NS - 1)
    def _zero_tail():
        pltpu.sync_copy(zero_hbm.at[pl.ds(_TAIL0, _TAILN)],
                        acc.at[pl.ds(_TAIL0, _TAILN)])

    pltpu.sync_copy(src_hbm.at[wid], src_v)
    pltpu.sync_copy(dst_hbm.at[wid], dst_v)
    plsc.subcore_barrier()

    def start_gather(j, buf, sem):
        pltpu.async_copy(g_hbm.at[src_v.at[j]], buf, sem)

    def wait_gather(buf, sem):
        pltpu.make_async_copy(g_hbm.at[src_v.at[0]], buf, sem).wait()

    def start_scatter(j, buf, sem):
        pltpu.async_copy(buf, acc.at[dst_v.at[j]], sem, add=True)

    def wait_scatter(buf, sem):
        pltpu.make_async_copy(buf, acc.at[dst_v.at[0]], sem).wait()

    for b in range(_NBUF):
        start_gather(b, rows[b], gsem[b])

    def body(i, carry):
        j = i * _NBUF
        for b in range(_NBUF):
            wait_gather(rows[b], gsem[b])
            start_scatter(j + b, rows[b], ssem[b])
        for b in range(_NBUF):
            wait_scatter(rows[b], ssem[b])

            @pl.when(j + _NBUF + b < _NCHUNK)
            def _g(b=b, j=j):
                start_gather(j + _NBUF + b, rows[b], gsem[b])

        return carry

    lax.fori_loop(0, _NCHUNK // _NBUF, body, 0)
    plsc.subcore_barrier()
    pltpu.sync_copy(acc.at[pl.ds(r0, _RPT)], out_hbm.at[cid, pl.ds(r0, _RPT)])

    @pl.when(sid == _NS - 1)
    def _out_tail():
        pltpu.sync_copy(acc.at[pl.ds(_TAIL0, _TAILN)],
                        out_hbm.at[cid, pl.ds(_TAIL0, _TAILN)])


_RB = 2000
_GRID = _N // _RB


def _prologue_body(x_ref, w1_ref, b1_ref, p_ref,
                   h_ref, g_ref, dinv_ref):
    deg = p_ref[0, :, :1] + p_ref[1, :, :1] + 1.0
    dinv = lax.rsqrt(deg)
    h = jnp.dot(x_ref[...], w1_ref[...],
                preferred_element_type=jnp.float32,
                precision=lax.Precision.HIGHEST) + b1_ref[...]
    h = jnp.maximum(h, 0.0)
    h_ref[...] = h
    g_ref[...] = dinv * h
    dinv_ref[...] = dinv


_prologue = pl.pallas_call(
    _prologue_body,
    grid=(_GRID,),
    in_specs=[
        pl.BlockSpec((_RB, _F), lambda i: (i, 0)),
        pl.BlockSpec((_F, _H), lambda i: (0, 0)),
        pl.BlockSpec((1, _H), lambda i: (0, 0)),
        pl.BlockSpec((_NC, _RB, _H), lambda i: (0, i, 0)),
    ],
    out_specs=[
        pl.BlockSpec((_RB, _H), lambda i: (i, 0)),
        pl.BlockSpec((_RB, _H), lambda i: (i, 0)),
        pl.BlockSpec((_RB, 1), lambda i: (i, 0)),
    ],
    out_shape=[
        jax.ShapeDtypeStruct((_N, _H), jnp.float32),
        jax.ShapeDtypeStruct((_N, _H), jnp.float32),
        jax.ShapeDtypeStruct((_N, 1), jnp.float32),
    ],
)


def _support(dinv_ref, s_ref, g_ref, h0_ref):
    ax = dinv_ref[...] * (s_ref[0] + s_ref[1] + g_ref[...])
    return (1.0 - _ALPHA) * ax + _ALPHA * h0_ref[...]


def _layer_body(dinv_ref, s_ref, g_ref, h0_ref, m_ref, gout_ref):
    h = jnp.dot(_support(dinv_ref, s_ref, g_ref, h0_ref), m_ref[...],
                preferred_element_type=jnp.float32,
                precision=lax.Precision.HIGHEST)
    gout_ref[...] = dinv_ref[...] * jnp.maximum(h, 0.0)


_layer = pl.pallas_call(
    _layer_body,
    grid=(_GRID,),
    in_specs=[
        pl.BlockSpec((_RB, 1), lambda i: (i, 0)),
        pl.BlockSpec((_NC, _RB, _H), lambda i: (0, i, 0)),
        pl.BlockSpec((_RB, _H), lambda i: (i, 0)),
        pl.BlockSpec((_RB, _H), lambda i: (i, 0)),
        pl.BlockSpec((_H, _H), lambda i: (0, 0)),
    ],
    out_specs=pl.BlockSpec((_RB, _H), lambda i: (i, 0)),
    out_shape=jax.ShapeDtypeStruct((_N, _H), jnp.float32),
)


def _final_body(dinv_ref, s_ref, g_ref, h0_ref, m_ref, w2_ref, b2_ref,
                out_ref):
    h = jnp.dot(_support(dinv_ref, s_ref, g_ref, h0_ref), m_ref[...],
                preferred_element_type=jnp.float32,
                precision=lax.Precision.HIGHEST)
    h = jnp.maximum(h, 0.0)
    out_ref[...] = jnp.dot(h, w2_ref[...],
                           preferred_element_type=jnp.float32,
                           precision=lax.Precision.HIGHEST) + b2_ref[...]


_final = pl.pallas_call(
    _final_body,
    grid=(_GRID,),
    in_specs=[
        pl.BlockSpec((_RB, 1), lambda i: (i, 0)),
        pl.BlockSpec((_NC, _RB, _H), lambda i: (0, i, 0)),
        pl.BlockSpec((_RB, _H), lambda i: (i, 0)),
        pl.BlockSpec((_RB, _H), lambda i: (i, 0)),
        pl.BlockSpec((_H, _H), lambda i: (0, 0)),
        pl.BlockSpec((_H, _H), lambda i: (0, 0)),
        pl.BlockSpec((1, _H), lambda i: (0, 0)),
    ],
    out_specs=pl.BlockSpec((_RB, _H), lambda i: (i, 0)),
    out_shape=jax.ShapeDtypeStruct((_N, _H), jnp.float32),
)


def kernel(x, edges, W1, b1, conv_w, W2, b2):
    src2 = edges[0].reshape(_NW, _NCHUNK, _CH)
    dst2 = edges[1].reshape(_NW, _NCHUNK, _CH)
    zeros = jnp.zeros((_N, _H), jnp.float32)
    ones = jnp.ones((_N, _H), jnp.float32)

    beta = np.log(_THETA / np.arange(1, _L + 1) + 1.0).astype(np.float32)
    eye = jnp.eye(_H, dtype=jnp.float32)
    M = (1.0 - beta)[:, None, None] * eye + beta[:, None, None] * conv_w

    p = _segsum_sc(ones, src2, dst2, zeros)
    h0, g, dinv = _prologue(x, W1, b1.reshape(1, _H), p)
    for l in range(_L - 1):
        s = _segsum_sc(g, src2, dst2, zeros)
        g = _layer(dinv, s, g, h0, M[l])
    s = _segsum_sc(g, src2, dst2, zeros)
    return _final(dinv, s, g, h0, M[_L - 1], W2, b2.reshape(1, _H))


# speedup vs baseline: 1.2954x; 1.2006x over previous
import functools
import numpy as np
import jax
import jax.numpy as jnp
from jax import lax
from jax.experimental import pallas as pl
from jax.experimental.pallas import tpu as pltpu
from jax.experimental.pallas import tpu_sc as plsc

_N = 10000
_E = 320000
_F = 128
_H = 64
_L = 64
_ALPHA = 0.1
_THETA = 0.6

_NC = 2
_NS = 16
_NW = _NC * _NS
_EPW = _E // _NW
_CH = 125
_NCHUNK = _EPW // _CH
_NBUF = 8
_RPT = 624
_TAIL0 = _NS * _RPT
_TAILN = _N - _TAIL0

_mesh = plsc.VectorSubcoreMesh(core_axis_name="c", subcore_axis_name="s")


@functools.partial(
    pl.kernel,
    out_type=jax.ShapeDtypeStruct((_NC, _N, _H), jnp.float32),
    mesh=_mesh,
    scratch_types=[
        pltpu.VMEM((_NCHUNK, _CH), jnp.int32),
        pltpu.VMEM((_NCHUNK, _CH), jnp.int32),
        [pltpu.VMEM((_CH, _H), jnp.float32) for _ in range(_NBUF)],
        pltpu.VMEM_SHARED((_N, _H), jnp.float32),
        [pltpu.SemaphoreType.DMA for _ in range(_NBUF)],
        [pltpu.SemaphoreType.DMA for _ in range(_NBUF)],
    ],
    compiler_params=pltpu.CompilerParams(use_tc_tiling_on_sc=False),
)
def _segsum_sc(g_hbm, src_hbm, dst_hbm, zero_hbm, out_hbm,
               src_v, dst_v, rows, acc, gsem, ssem):
    cid = lax.axis_index("c")
    sid = lax.axis_index("s")
    wid = sid * _NC + cid
    r0 = sid * _RPT
    pltpu.sync_copy(zero_hbm.at[pl.ds(r0, _RPT)], acc.at[pl.ds(r0, _RPT)])

    @pl.when(sid == _NS - 1)
    def _zero_tail():
        pltpu.sync_copy(zero_hbm.at[pl.ds(_TAIL0, _TAILN)],
                        acc.at[pl.ds(_TAIL0, _TAILN)])

    pltpu.sync_copy(src_hbm.at[wid], src_v)
    pltpu.sync_copy(dst_hbm.at[wid], dst_v)
    plsc.subcore_barrier()

    def start_gather(j, buf, sem):
        pltpu.async_copy(g_hbm.at[src_v.at[j]], buf, sem)

    def wait_gather(buf, sem):
        pltpu.make_async_copy(g_hbm.at[src_v.at[0]], buf, sem).wait()

    def start_scatter(j, buf, sem):
        pltpu.async_copy(buf, acc.at[dst_v.at[j]], sem, add=True)

    def wait_scatter(buf, sem):
        pltpu.make_async_copy(buf, acc.at[dst_v.at[0]], sem).wait()

    for b in range(_NBUF):
        start_gather(b, rows[b], gsem[b])

    def body(i, carry):
        j = i * _NBUF
        for b in range(_NBUF):
            wait_gather(rows[b], gsem[b])
            start_scatter(j + b, rows[b], ssem[b])
        for b in range(_NBUF):
            wait_scatter(rows[b], ssem[b])

            @pl.when(j + _NBUF + b < _NCHUNK)
            def _g(b=b, j=j):
                start_gather(j + _NBUF + b, rows[b], gsem[b])

        return carry

    lax.fori_loop(0, _NCHUNK // _NBUF, body, 0)
    plsc.subcore_barrier()
    pltpu.sync_copy(acc.at[pl.ds(r0, _RPT)], out_hbm.at[cid, pl.ds(r0, _RPT)])

    @pl.when(sid == _NS - 1)
    def _out_tail():
        pltpu.sync_copy(acc.at[pl.ds(_TAIL0, _TAILN)],
                        out_hbm.at[cid, pl.ds(_TAIL0, _TAILN)])


_RB = 2000
_GRID = _N // _RB


def _prologue_body(x_ref, w1_ref, b1_ref, p_ref,
                   h_ref, g_ref, dinv_ref):
    deg = p_ref[0, :, :1] + p_ref[1, :, :1] + 1.0
    dinv = lax.rsqrt(deg)
    h = jnp.dot(x_ref[...], w1_ref[...],
                preferred_element_type=jnp.float32,
                precision=lax.Precision.HIGHEST) + b1_ref[...]
    h = jnp.maximum(h, 0.0)
    h_ref[...] = h
    g_ref[...] = dinv * h
    dinv_ref[...] = dinv


_prologue = pl.pallas_call(
    _prologue_body,
    grid=(_GRID,),
    in_specs=[
        pl.BlockSpec((_RB, _F), lambda i: (i, 0)),
        pl.BlockSpec((_F, _H), lambda i: (0, 0)),
        pl.BlockSpec((1, _H), lambda i: (0, 0)),
        pl.BlockSpec((_NC, _RB, _H), lambda i: (0, i, 0)),
    ],
    out_specs=[
        pl.BlockSpec((_RB, _H), lambda i: (i, 0)),
        pl.BlockSpec((_RB, _H), lambda i: (i, 0)),
        pl.BlockSpec((_RB, 1), lambda i: (i, 0)),
    ],
    out_shape=[
        jax.ShapeDtypeStruct((_N, _H), jnp.float32),
        jax.ShapeDtypeStruct((_N, _H), jnp.float32),
        jax.ShapeDtypeStruct((_N, 1), jnp.float32),
    ],
)


def _support(dinv_ref, s_ref, g_ref, h0_ref):
    ax = dinv_ref[...] * (s_ref[0] + s_ref[1] + g_ref[...])
    return (1.0 - _ALPHA) * ax + _ALPHA * h0_ref[...]


def _layer_body(dinv_ref, s_ref, g_ref, h0_ref, m_ref, gout_ref):
    h = jnp.dot(_support(dinv_ref, s_ref, g_ref, h0_ref), m_ref[...],
                preferred_element_type=jnp.float32,
                precision=lax.Precision.HIGHEST)
    gout_ref[...] = dinv_ref[...] * jnp.maximum(h, 0.0)


_layer = pl.pallas_call(
    _layer_body,
    grid=(_GRID,),
    in_specs=[
        pl.BlockSpec((_RB, 1), lambda i: (i, 0)),
        pl.BlockSpec((_NC, _RB, _H), lambda i: (0, i, 0)),
        pl.BlockSpec((_RB, _H), lambda i: (i, 0)),
        pl.BlockSpec((_RB, _H), lambda i: (i, 0)),
        pl.BlockSpec((_H, _H), lambda i: (0, 0)),
    ],
    out_specs=pl.BlockSpec((_RB, _H), lambda i: (i, 0)),
    out_shape=jax.ShapeDtypeStruct((_N, _H), jnp.float32),
)


def _final_body(dinv_ref, s_ref, g_ref, h0_ref, m_ref, w2_ref, b2_ref,
                out_ref):
    h = jnp.dot(_support(dinv_ref, s_ref, g_ref, h0_ref), m_ref[...],
                preferred_element_type=jnp.float32,
                precision=lax.Precision.HIGHEST)
    h = jnp.maximum(h, 0.0)
    out_ref[...] = jnp.dot(h, w2_ref[...],
                           preferred_element_type=jnp.float32,
                           precision=lax.Precision.HIGHEST) + b2_ref[...]


_final = pl.pallas_call(
    _final_body,
    grid=(_GRID,),
    in_specs=[
        pl.BlockSpec((_RB, 1), lambda i: (i, 0)),
        pl.BlockSpec((_NC, _RB, _H), lambda i: (0, i, 0)),
        pl.BlockSpec((_RB, _H), lambda i: (i, 0)),
        pl.BlockSpec((_RB, _H), lambda i: (i, 0)),
        pl.BlockSpec((_H, _H), lambda i: (0, 0)),
        pl.BlockSpec((_H, _H), lambda i: (0, 0)),
        pl.BlockSpec((1, _H), lambda i: (0, 0)),
    ],
    out_specs=pl.BlockSpec((_RB, _H), lambda i: (i, 0)),
    out_shape=jax.ShapeDtypeStruct((_N, _H), jnp.float32),
)


def kernel(x, edges, W1, b1, conv_w, W2, b2):
    src2 = edges[0].reshape(_NW, _NCHUNK, _CH)
    dst2 = edges[1].reshape(_NW, _NCHUNK, _CH)
    zeros = jnp.zeros((_N, _H), jnp.float32)
    ones = jnp.ones((_N, _H), jnp.float32)

    beta = np.log(_THETA / np.arange(1, _L + 1) + 1.0).astype(np.float32)
    eye = jnp.eye(_H, dtype=jnp.float32)
    M = (1.0 - beta)[:, None, None] * eye + beta[:, None, None] * conv_w

    p = _segsum_sc(ones, src2, dst2, zeros)
    h0, g, dinv = _prologue(x, W1, b1.reshape(1, _H), p)
    for l in range(_L - 1):
        s = _segsum_sc(g, src2, dst2, zeros)
        g = _layer(dinv, s, g, h0, M[l])
    s = _segsum_sc(g, src2, dst2, zeros)
    return _final(dinv, s, g, h0, M[_L - 1], W2, b2.reshape(1, _H))
